# Initial kernel scaffold; baseline (speedup 1.0000x reference)
import jax
import jax.numpy as jnp
from jax.experimental import pallas as pl


def kernel(x, edge_index, global_features, batch, W0, b0, W1, b1, W2, b2, Wf0, bf0, Wf1, bf1, Wo, bo):
    raise NotImplementedError("write your pallas kernel here")


# speedup vs baseline: 16.2103x; 16.2103x over previous
import functools

import jax
import jax.numpy as jnp
from jax import lax
from jax.experimental import pallas as pl
from jax.experimental.pallas import tpu as pltpu
from jax.experimental.pallas import tpu_sc as plsc

N = 10000
NP = 10240
E = 320000
D = 128
G = 16
NC = 2
NS = 16
K = 80
CH = E // (NC * NS * K)
RPT = NP // NS


def _deg_body(dst_hbm, deg_out, dstv, onesv, zbuf, acc):
    cid = lax.axis_index("c")
    sid = lax.axis_index("s")
    for i in range(RPT // 16):
        zbuf[pl.ds(16 * i, 16)] = jnp.zeros((16,), jnp.float32)
    for i in range(K // 16):
        onesv[pl.ds(16 * i, 16)] = jnp.ones((16,), jnp.float32)
    pltpu.sync_copy(zbuf, acc.at[pl.ds(sid * RPT, RPT)])
    pltpu.sync_copy(dst_hbm.at[cid, sid], dstv)
    plsc.subcore_barrier()

    def chunk(j, carry):
        pltpu.sync_copy(onesv, acc.at[dstv.at[j]], add=True)
        return carry

    lax.fori_loop(0, CH, chunk, 0)
    plsc.subcore_barrier()
    pltpu.sync_copy(acc.at[pl.ds(sid * RPT, RPT)],
                    deg_out.at[pl.ds(cid * NP + sid * RPT, RPT)])


@functools.cache
def _deg_call():
    mesh = plsc.VectorSubcoreMesh(core_axis_name="c", subcore_axis_name="s",
                                  num_cores=NC, num_subcores=NS)
    return pl.kernel(
        _deg_body,
        out_type=jax.ShapeDtypeStruct((NC * NP,), jnp.float32),
        mesh=mesh,
        scratch_types=[
            pltpu.VMEM((CH, K), jnp.int32),
            pltpu.VMEM((K,), jnp.float32),
            pltpu.VMEM((RPT,), jnp.float32),
            pltpu.VMEM_SHARED((NP,), jnp.float32),
        ],
    )


def _agg_body(h_hbm, src_hbm, dst_hbm, out_hbm, srcv, dstv, rows, sem, acc):
    cid = lax.axis_index("c")
    sid = lax.axis_index("s")
    r0 = sid * RPT
    pltpu.sync_copy(h_hbm.at[pl.ds(r0, RPT)], acc.at[pl.ds(r0, RPT)])
    pltpu.sync_copy(src_hbm.at[cid, sid], srcv)
    pltpu.sync_copy(dst_hbm.at[cid, sid], dstv)
    plsc.subcore_barrier()

    def chunk(j, carry):
        pltpu.async_copy(h_hbm.at[srcv.at[j]], rows, sem).wait()
        pltpu.sync_copy(rows, acc.at[dstv.at[j]], add=True)
        return carry

    lax.fori_loop(0, CH, chunk, 0)
    plsc.subcore_barrier()
    pltpu.sync_copy(acc.at[pl.ds(r0, RPT)], out_hbm.at[cid, pl.ds(r0, RPT)])


@functools.cache
def _agg_call():
    mesh = plsc.VectorSubcoreMesh(core_axis_name="c", subcore_axis_name="s",
                                  num_cores=NC, num_subcores=NS)
    return pl.kernel(
        _agg_body,
        out_type=jax.ShapeDtypeStruct((NC, NP, D), jnp.float32),
        mesh=mesh,
        scratch_types=[
            pltpu.VMEM((CH, K), jnp.int32),
            pltpu.VMEM((CH, K), jnp.int32),
            pltpu.VMEM((K, D), jnp.float32),
            pltpu.SemaphoreType.DMA,
            pltpu.VMEM_SHARED((NP, D), jnp.float32),
        ],
    )


BR = 640


def _first_body(degp_ref, x_ref, w_ref, hp_ref, dinv_ref):
    r0 = pl.program_id(0) * BR
    deg = (degp_ref[pl.ds(r0, BR), :] + degp_ref[pl.ds(NP + r0, BR), :] + 1.0)
    dinv = lax.rsqrt(deg)
    g = jnp.dot(x_ref[...], w_ref[...], preferred_element_type=jnp.float32)
    hp_ref[...] = g * dinv
    dinv_ref[...] = dinv


_first_call = pl.pallas_call(
    _first_body,
    grid=(NP // BR,),
    in_specs=[
        pl.BlockSpec((NC * NP, 1), lambda i: (0, 0)),
        pl.BlockSpec((BR, D), lambda i: (i, 0)),
        pl.BlockSpec((D, D), lambda i: (0, 0)),
    ],
    out_specs=[
        pl.BlockSpec((BR, D), lambda i: (i, 0)),
        pl.BlockSpec((BR, 1), lambda i: (i, 0)),
    ],
    out_shape=[
        jax.ShapeDtypeStruct((NP, D), jnp.float32),
        jax.ShapeDtypeStruct((NP, 1), jnp.float32),
    ],
)


def _lrelu(v):
    return jnp.where(v >= 0, v, 0.01 * v)


def _mid_body(agg_ref, hp_ref, dinv_ref, b_ref, w_ref, out_ref):
    s = agg_ref[0] + agg_ref[1] - hp_ref[...]
    act = _lrelu(dinv_ref[...] * s + b_ref[...])
    g = jnp.dot(act, w_ref[...], preferred_element_type=jnp.float32)
    out_ref[...] = g * dinv_ref[...]


_mid_call = pl.pallas_call(
    _mid_body,
    grid=(NP // BR,),
    in_specs=[
        pl.BlockSpec((NC, BR, D), lambda i: (0, i, 0)),
        pl.BlockSpec((BR, D), lambda i: (i, 0)),
        pl.BlockSpec((BR, 1), lambda i: (i, 0)),
        pl.BlockSpec((1, D), lambda i: (0, 0)),
        pl.BlockSpec((D, D), lambda i: (0, 0)),
    ],
    out_specs=pl.BlockSpec((BR, D), lambda i: (i, 0)),
    out_shape=jax.ShapeDtypeStruct((NP, D), jnp.float32),
)


def _final_body(agg_ref, hp_ref, dinv_ref, b_ref, batch_ref, gf_ref,
                wf0_ref, bf0_ref, wf1_ref, bf1_ref, wo_ref, bo_ref, out_ref):
    s = agg_ref[0] + agg_ref[1] - hp_ref[...]
    h = _lrelu(dinv_ref[...] * s + b_ref[...])
    batch = batch_ref[...]
    onehot_t = (batch ==
                lax.broadcasted_iota(jnp.int32, (NP, G), 1)).astype(jnp.float32)
    dn = (((0,), (0,)), ((), ()))
    ssum = lax.dot_general(onehot_t, h, dn,
                           preferred_element_type=jnp.float32)
    counts = lax.dot_general(onehot_t, jnp.ones((NP, 1), jnp.float32), dn,
                             preferred_element_type=jnp.float32)
    mean = ssum / jnp.maximum(counts, 1.0)
    neg = jnp.float32(-jnp.inf)
    cols = []
    for g in range(G):
        cols.append(jnp.max(jnp.where(batch == g, h, neg), axis=0,
                            keepdims=True))
    maxp = jnp.concatenate(cols, axis=0)
    maxp = jnp.where(jnp.isfinite(maxp), maxp, 0.0)
    z = jnp.concatenate([mean, maxp, gf_ref[...]], axis=1)
    z = _lrelu(jnp.dot(z, wf0_ref[...], preferred_element_type=jnp.float32)
               + bf0_ref[...])
    z = _lrelu(jnp.dot(z, wf1_ref[...], preferred_element_type=jnp.float32)
               + bf1_ref[...])
    out_ref[...] = (jnp.dot(z, wo_ref[...], preferred_element_type=jnp.float32)
                    + bo_ref[...])


def _final_call(agg, hp, dinv, b, batch, gf, wf0, bf0, wf1, bf1, wo, bo):
    return pl.pallas_call(
        _final_body,
        out_shape=jax.ShapeDtypeStruct((G, 1), jnp.float32),
    )(agg, hp, dinv, b, batch, gf, wf0, bf0, wf1, bf1, wo, bo)


@jax.jit
def kernel(x, edge_index, global_features, batch,
           W0, b0, W1, b1, W2, b2, Wf0, bf0, Wf1, bf1, Wo, bo):
    src = edge_index[0].reshape(NC, NS, CH, K)
    dst = edge_index[1].reshape(NC, NS, CH, K)
    x_p = jnp.pad(x, ((0, NP - N), (0, 0)))
    batch_p = jnp.pad(batch, (0, NP - N), constant_values=G).reshape(NP, 1)
    degp = _deg_call()(dst).reshape(NC * NP, 1)
    hp0, dinv = _first_call(degp, x_p, W0)
    agg0 = _agg_call()(hp0, src, dst)
    hp1 = _mid_call(agg0, hp0, dinv, b0.reshape(1, D), W1)
    agg1 = _agg_call()(hp1, src, dst)
    hp2 = _mid_call(agg1, hp1, dinv, b1.reshape(1, D), W2)
    agg2 = _agg_call()(hp2, src, dst)
    return _final_call(agg2, hp2, dinv, b2.reshape(1, D), batch_p,
                       global_features, Wf0, bf0.reshape(1, -1),
                       Wf1, bf1.reshape(1, -1), Wo, bo.reshape(1, -1))


# speedup vs baseline: 23.1625x; 1.4289x over previous
import functools

import jax
import jax.numpy as jnp
from jax import lax
from jax.experimental import pallas as pl
from jax.experimental.pallas import tpu as pltpu
from jax.experimental.pallas import tpu_sc as plsc

N = 10000
NP = 10240
E = 320000
D = 128
G = 16
NC = 2
NS = 16
K = 80
CH = E // (NC * NS * K)
RPT = NP // NS


def _deg_body(dst_hbm, deg_out, dstv, onesv, zbuf, acc):
    cid = lax.axis_index("c")
    sid = lax.axis_index("s")
    for i in range(RPT // 16):
        zbuf[pl.ds(16 * i, 16)] = jnp.zeros((16,), jnp.float32)
    for i in range(K // 16):
        onesv[pl.ds(16 * i, 16)] = jnp.ones((16,), jnp.float32)
    pltpu.sync_copy(zbuf, acc.at[pl.ds(sid * RPT, RPT)])
    pltpu.sync_copy(dst_hbm.at[cid, sid], dstv)
    plsc.subcore_barrier()

    def chunk(j, carry):
        pltpu.sync_copy(onesv, acc.at[dstv.at[j]], add=True)
        return carry

    lax.fori_loop(0, CH, chunk, 0)
    plsc.subcore_barrier()
    pltpu.sync_copy(acc.at[pl.ds(sid * RPT, RPT)],
                    deg_out.at[pl.ds(cid * NP + sid * RPT, RPT)])


@functools.cache
def _deg_call():
    mesh = plsc.VectorSubcoreMesh(core_axis_name="c", subcore_axis_name="s",
                                  num_cores=NC, num_subcores=NS)
    return pl.kernel(
        _deg_body,
        out_type=jax.ShapeDtypeStruct((NC * NP,), jnp.float32),
        mesh=mesh,
        scratch_types=[
            pltpu.VMEM((CH, K), jnp.int32),
            pltpu.VMEM((K,), jnp.float32),
            pltpu.VMEM((RPT,), jnp.float32),
            pltpu.VMEM_SHARED((NP,), jnp.float32),
        ],
    )


def _agg_body(h_hbm, src_hbm, dst_hbm, out_hbm, sidx, didx,
              rows0, rows1, rows2, g0, g1, g2, i0, i1, i2, d0, d1, d2, acc):
    rows = (rows0, rows1, rows2)
    gs = (g0, g1, g2)
    isems = (i0, i1, i2)
    dsems = (d0, d1, d2)
    cid = lax.axis_index("c")
    sid = lax.axis_index("s")
    r0 = sid * RPT
    base = (cid * NS + sid) * (CH * K)
    pltpu.sync_copy(src_hbm.at[pl.ds(base, K)], sidx.at[0])
    pltpu.sync_copy(src_hbm.at[pl.ds(base + K, K)], sidx.at[1])
    pltpu.async_copy(h_hbm.at[sidx.at[0]], rows[0], gs[0])
    pltpu.async_copy(h_hbm.at[sidx.at[1]], rows[1], gs[1])
    pltpu.async_copy(src_hbm.at[pl.ds(base + 2 * K, K)], sidx.at[2], isems[2])
    for b in range(3):
        pltpu.sync_copy(dst_hbm.at[pl.ds(base + b * K, K)], didx.at[b])
    pltpu.sync_copy(h_hbm.at[pl.ds(r0, RPT)], acc.at[pl.ds(r0, RPT)])
    plsc.subcore_barrier()

    def group(g, carry):
        for b in range(3):
            j = 3 * g + b
            f = (b + 2) % 3
            pltpu.make_async_copy(h_hbm.at[sidx.at[b]], rows[b],
                                  gs[b]).wait()
            pltpu.make_async_copy(src_hbm.at[pl.ds(base + j * K, K)], sidx.at[f],
                                  isems[f]).wait()
            pltpu.async_copy(h_hbm.at[sidx.at[f]], rows[f], gs[f])
            @pl.when(g >= 1)
            def _():
                pltpu.make_async_copy(dst_hbm.at[pl.ds(base + j * K, K)], didx.at[b],
                                      dsems[b]).wait()
            pltpu.sync_copy(rows[b], acc.at[didx.at[b]], add=True)
            if b < 2:
                pltpu.async_copy(src_hbm.at[pl.ds(base + (j + 3) * K, K)], sidx.at[b],
                                 isems[b])
                pltpu.async_copy(dst_hbm.at[pl.ds(base + (j + 3) * K, K)], didx.at[b],
                                 dsems[b])
            else:
                @pl.when(g < (CH - 3) // 3)
                def _():
                    pltpu.async_copy(src_hbm.at[pl.ds(base + (j + 3) * K, K)], sidx.at[b],
                                     isems[b])
                    pltpu.async_copy(dst_hbm.at[pl.ds(base + (j + 3) * K, K)], didx.at[b],
                                     dsems[b])
        return carry

    lax.fori_loop(0, (CH - 2) // 3, group, 0)
    for b, j in ((0, CH - 2), (1, CH - 1)):
        pltpu.make_async_copy(h_hbm.at[sidx.at[b]], rows[b], gs[b]).wait()
        pltpu.make_async_copy(dst_hbm.at[pl.ds(base + j * K, K)], didx.at[b],
                              dsems[b]).wait()
        pltpu.sync_copy(rows[b], acc.at[didx.at[b]], add=True)
    plsc.subcore_barrier()
    pltpu.sync_copy(acc.at[pl.ds(r0, RPT)], out_hbm.at[cid, pl.ds(r0, RPT)])


@functools.cache
def _agg_call():
    mesh = plsc.VectorSubcoreMesh(core_axis_name="c", subcore_axis_name="s",
                                  num_cores=NC, num_subcores=NS)
    return pl.kernel(
        _agg_body,
        out_type=jax.ShapeDtypeStruct((NC, NP, D), jnp.float32),
        mesh=mesh,
        scratch_types=[
            pltpu.VMEM((3, K), jnp.int32),
            pltpu.VMEM((3, K), jnp.int32),
            pltpu.VMEM((K, D), jnp.float32),
            pltpu.VMEM((K, D), jnp.float32),
            pltpu.VMEM((K, D), jnp.float32),
        ] + [pltpu.SemaphoreType.DMA] * 9 + [
            pltpu.VMEM_SHARED((NP, D), jnp.float32),
        ],
    )


BR = 640


def _first_body(degp_ref, x_ref, w_ref, hp_ref, dinv_ref):
    r0 = pl.program_id(0) * BR
    deg = (degp_ref[pl.ds(r0, BR), :] + degp_ref[pl.ds(NP + r0, BR), :] + 1.0)
    dinv = lax.rsqrt(deg)
    valid = (lax.broadcasted_iota(jnp.int32, (BR, 1), 0) + r0) < N
    xv = jnp.where(valid, x_ref[...], 0.0)
    g = jnp.dot(xv, w_ref[...], preferred_element_type=jnp.float32)
    hp_ref[...] = g * dinv
    dinv_ref[...] = dinv


_first_call = pl.pallas_call(
    _first_body,
    grid=(NP // BR,),
    in_specs=[
        pl.BlockSpec((NC * NP, 1), lambda i: (0, 0)),
        pl.BlockSpec((BR, D), lambda i: (i, 0)),
        pl.BlockSpec((D, D), lambda i: (0, 0)),
    ],
    out_specs=[
        pl.BlockSpec((BR, D), lambda i: (i, 0)),
        pl.BlockSpec((BR, 1), lambda i: (i, 0)),
    ],
    out_shape=[
        jax.ShapeDtypeStruct((NP, D), jnp.float32),
        jax.ShapeDtypeStruct((NP, 1), jnp.float32),
    ],
)


def _lrelu(v):
    return jnp.where(v >= 0, v, 0.01 * v)


def _mid_body(agg_ref, hp_ref, dinv_ref, b_ref, w_ref, out_ref):
    s = agg_ref[0] + agg_ref[1] - hp_ref[...]
    act = _lrelu(dinv_ref[...] * s + b_ref[...])
    g = jnp.dot(act, w_ref[...], preferred_element_type=jnp.float32)
    out_ref[...] = g * dinv_ref[...]


_mid_call = pl.pallas_call(
    _mid_body,
    grid=(NP // BR,),
    in_specs=[
        pl.BlockSpec((NC, BR, D), lambda i: (0, i, 0)),
        pl.BlockSpec((BR, D), lambda i: (i, 0)),
        pl.BlockSpec((BR, 1), lambda i: (i, 0)),
        pl.BlockSpec((1, D), lambda i: (0, 0)),
        pl.BlockSpec((D, D), lambda i: (0, 0)),
    ],
    out_specs=pl.BlockSpec((BR, D), lambda i: (i, 0)),
    out_shape=jax.ShapeDtypeStruct((NP, D), jnp.float32),
)


def _final_body(agg_ref, hp_ref, dinv_ref, b_ref, batch_ref, gf_ref,
                wf0_ref, bf0_ref, wf1_ref, bf1_ref, wo_ref, bo_ref, out_ref):
    s = agg_ref[0] + agg_ref[1] - hp_ref[...]
    h = _lrelu(dinv_ref[...] * s + b_ref[...])
    valid = lax.broadcasted_iota(jnp.int32, (NP, 1), 0) < N
    batch = jnp.where(valid, batch_ref[...], G)
    onehot_t = (batch ==
                lax.broadcasted_iota(jnp.int32, (NP, G), 1)).astype(jnp.float32)
    dn = (((0,), (0,)), ((), ()))
    ssum = lax.dot_general(onehot_t, h, dn,
                           preferred_element_type=jnp.float32)
    counts = lax.dot_general(onehot_t, jnp.ones((NP, 1), jnp.float32), dn,
                             preferred_element_type=jnp.float32)
    mean = ssum / jnp.maximum(counts, 1.0)
    neg = jnp.float32(-jnp.inf)
    cols = []
    for g in range(G):
        cols.append(jnp.max(jnp.where(batch == g, h, neg), axis=0,
                            keepdims=True))
    maxp = jnp.concatenate(cols, axis=0)
    maxp = jnp.where(jnp.isfinite(maxp), maxp, 0.0)
    z = jnp.concatenate([mean, maxp, gf_ref[...]], axis=1)
    z = _lrelu(jnp.dot(z, wf0_ref[...], preferred_element_type=jnp.float32)
               + bf0_ref[...])
    z = _lrelu(jnp.dot(z, wf1_ref[...], preferred_element_type=jnp.float32)
               + bf1_ref[...])
    out_ref[...] = (jnp.dot(z, wo_ref[...], preferred_element_type=jnp.float32)
                    + bo_ref[...])


def _final_call(agg, hp, dinv, b, batch, gf, wf0, bf0, wf1, bf1, wo, bo):
    def _full(a):
        n = len(a.shape)
        return pl.BlockSpec(a.shape, lambda i, n=n: (0,) * n)
    specs = [_full(a) for a in (agg, hp, dinv, b)]
    specs.append(pl.BlockSpec((NP, 1), lambda i: (0, 0)))
    specs += [_full(a) for a in (gf, wf0, bf0, wf1, bf1, wo, bo)]
    return pl.pallas_call(
        _final_body,
        grid=(1,),
        in_specs=specs,
        out_specs=pl.BlockSpec((G, 1), lambda i: (0, 0)),
        out_shape=jax.ShapeDtypeStruct((G, 1), jnp.float32),
    )(agg, hp, dinv, b, batch, gf, wf0, bf0, wf1, bf1, wo, bo)


@jax.jit
def kernel(x, edge_index, global_features, batch,
           W0, b0, W1, b1, W2, b2, Wf0, bf0, Wf1, bf1, Wo, bo):
    dst4 = edge_index[1].reshape(NC, NS, CH, K)
    src = edge_index[0]
    dst = edge_index[1]
    batch_p = batch.reshape(N, 1)
    degp = _deg_call()(dst4).reshape(NC * NP, 1)
    hp0, dinv = _first_call(degp, x, W0)
    agg0 = _agg_call()(hp0, src, dst)
    hp1 = _mid_call(agg0, hp0, dinv, b0.reshape(1, D), W1)
    agg1 = _agg_call()(hp1, src, dst)
    hp2 = _mid_call(agg1, hp1, dinv, b1.reshape(1, D), W2)
    agg2 = _agg_call()(hp2, src, dst)
    return _final_call(agg2, hp2, dinv, b2.reshape(1, D), batch_p,
                       global_features, Wf0, bf0.reshape(1, -1),
                       Wf1, bf1.reshape(1, -1), Wo, bo.reshape(1, -1))
